# baseline (speedup 1.0000x reference)
import functools

import jax
import jax.numpy as jnp
from jax import lax
from jax.experimental import pallas as pl
from jax.experimental.pallas import tpu as pltpu
from jax.experimental.pallas import tpu_sc as plsc

B = 8
NV = 4096
NG = 256
NW = 32
RPW = NV // NW
CH = 16
NCHUNK = RPW // CH
EPS = 1e-6


def _prep_body(pmap_ref, dis_ref, pm_ref, q_ref, nest_ref, m_ref):
    i = pl.program_id(0)
    nsteps = pl.num_programs(0)
    blockmax = jnp.max(dis_ref[...])
    prev = jnp.where(i == 0, -jnp.inf, m_ref[0, 0])
    cur = jnp.maximum(prev, blockmax)
    m_ref[0, 0] = cur

    @pl.when(i == nsteps - 1)
    def _():
        fp = jnp.sqrt(jnp.sum(pmap_ref[...] * pmap_ref[...], axis=2))
        pmax = jnp.max(fp, axis=1, keepdims=True)
        pm = fp / pmax
        pm_ref[...] = pm
        q_ref[...] = (1.0 - pm) * cur + EPS
        nest_ref[...] = jnp.sum(pm, axis=1)[None, :]


def _prep(prob_map, dis_matrix):
    blk = 512
    grid = NV // blk
    return pl.pallas_call(
        _prep_body,
        grid=(grid,),
        compiler_params=pltpu.CompilerParams(
            vmem_limit_bytes=100 * 1024 * 1024),
        in_specs=[
            pl.BlockSpec((B, NV, 4), lambda i: (0, 0, 0)),
            pl.BlockSpec((blk, NV), lambda i: (i, 0)),
        ],
        out_specs=[
            pl.BlockSpec((B, NV), lambda i: (0, 0)),
            pl.BlockSpec((B, NV), lambda i: (0, 0)),
            pl.BlockSpec((1, B), lambda i: (0, 0)),
            pl.BlockSpec((1, 1), lambda i: (0, 0), memory_space=pltpu.SMEM),
        ],
        out_shape=[
            jax.ShapeDtypeStruct((B, NV), jnp.float32),
            jax.ShapeDtypeStruct((B, NV), jnp.float32),
            jax.ShapeDtypeStruct((1, B), jnp.float32),
            jax.ShapeDtypeStruct((1, 1), jnp.float32),
        ],
    )(prob_map, dis_matrix)


_MESH = plsc.VectorSubcoreMesh(core_axis_name="c", subcore_axis_name="s")


@functools.partial(
    pl.kernel,
    mesh=_MESH,
    compiler_params=pltpu.CompilerParams(needs_layout_passes=False),
    out_type=[
        jax.ShapeDtypeStruct((NW, B * NG), jnp.float32),
        jax.ShapeDtypeStruct((NW, B * 16), jnp.float32),
    ],
    scratch_types=[
        pltpu.VMEM((CH * NV,), jnp.float32),
        pltpu.VMEM((B * NG,), jnp.int32),
        pltpu.VMEM((B * RPW,), jnp.float32),
        pltpu.VMEM((B * RPW,), jnp.float32),
        pltpu.VMEM((B * NG,), jnp.float32),
        pltpu.VMEM((B * 16,), jnp.float32),
    ],
)
def _scmain(dis_hbm, gt_hbm, pm_hbm, q_hbm, colsum_out, t1_out,
            rowbuf, idxbuf, pmbuf, qbuf, colsum, t1buf):
    c = lax.axis_index("c")
    s = lax.axis_index("s")
    wid = c * 16 + s
    r0 = wid * RPW

    pltpu.sync_copy(gt_hbm, idxbuf)
    for b in range(B):
        pltpu.sync_copy(pm_hbm.at[pl.ds(b * NV + r0, RPW)],
                        pmbuf.at[pl.ds(b * RPW, RPW)])
        pltpu.sync_copy(q_hbm.at[pl.ds(b * NV + r0, RPW)],
                        qbuf.at[pl.ds(b * RPW, RPW)])

    zero16 = jnp.zeros((16,), jnp.float32)
    for j in range(B * NG // 16):
        colsum[pl.ds(j * 16, 16)] = zero16

    def chunk_body(ci, t1c):
        pltpu.sync_copy(dis_hbm.at[pl.ds((r0 + ci * CH) * NV, CH * NV)], rowbuf)
        t1l = list(t1c)
        for b in range(B):
            cvecs = [idxbuf[pl.ds(b * NG + k * 16, 16)] for k in range(NG // 16)]

            def r_body(r, acc, b=b, cvecs=cvecs, ci=ci):
                lrow = ci * CH + r
                rowoff = jnp.full((16,), r * NV, dtype=jnp.int32)
                pmoff = jnp.full((16,), b * RPW + lrow, dtype=jnp.int32)
                pmv = plsc.load_gather(pmbuf, [pmoff])
                qv = plsc.load_gather(qbuf, [pmoff])
                minacc = jnp.full((16,), jnp.inf, dtype=jnp.float32)
                for k in range(NG // 16):
                    g = plsc.load_gather(rowbuf, [rowoff + cvecs[k]])
                    minacc = jnp.minimum(minacc, g)
                    rec = 1.0 / (g * pmv + qv)
                    plsc.addupdate(colsum.at[pl.ds(b * NG + k * 16, 16)], rec)
                rowmin = jnp.min(minacc)
                return acc + pmv * rowmin

            t1l[b] = lax.fori_loop(0, CH, r_body, t1l[b])
        return tuple(t1l)

    t1 = lax.fori_loop(0, NCHUNK, chunk_body,
                       tuple(jnp.zeros((16,), jnp.float32) for _ in range(B)))

    for b in range(B):
        t1buf[pl.ds(b * 16, 16)] = t1[b] * (1.0 / 16.0)

    pltpu.sync_copy(colsum, colsum_out.at[wid])
    pltpu.sync_copy(t1buf, t1_out.at[wid])


def _epi_body(colsum_ref, t1_ref, nest_ref, out_ref):
    S = jnp.sum(colsum_ref[...], axis=0, keepdims=True)
    term2_sum = jnp.sum(float(NV) / S) * (1.0 / NG)
    t1acc = 0.0
    for b in range(B):
        t1num_b = jnp.sum(t1_ref[:, b * 16:(b + 1) * 16])
        t1acc = t1acc + t1num_b / (nest_ref[0, b] + EPS)
    out_ref[0, 0] = (t1acc + term2_sum) * (1.0 / B)


def _epi(colsum_part, t1_part, nest):
    return pl.pallas_call(
        _epi_body,
        in_specs=[
            pl.BlockSpec(memory_space=pltpu.VMEM),
            pl.BlockSpec(memory_space=pltpu.VMEM),
            pl.BlockSpec(memory_space=pltpu.SMEM),
        ],
        out_specs=pl.BlockSpec(memory_space=pltpu.SMEM),
        out_shape=jax.ShapeDtypeStruct((1, 1), jnp.float32),
    )(colsum_part, t1_part, nest)


def kernel(prob_map, gt, dis_matrix):
    pm, q, nest, _m = _prep(prob_map, dis_matrix)
    gt_flat = gt.reshape(-1)
    colsum_part, t1_part = _scmain(dis_matrix.reshape(-1), gt_flat,
                                   pm.reshape(-1), q.reshape(-1))
    res = _epi(colsum_part, t1_part, nest)
    return res[0, 0]


# speedup vs baseline: 1.0332x; 1.0332x over previous
import functools

import jax
import jax.numpy as jnp
from jax import lax
from jax.experimental import pallas as pl
from jax.experimental.pallas import tpu as pltpu
from jax.experimental.pallas import tpu_sc as plsc

B = 8
NV = 4096
NG = 256
NW = 32
RPW = NV // NW
CH = 16
NCHUNK = RPW // CH
EPS = 1e-6


def _prep_body(pmap_ref, dis_ref, pm_ref, q_ref, nest_ref, m_ref):
    i = pl.program_id(0)
    nsteps = pl.num_programs(0)
    blockmax = jnp.max(dis_ref[...])
    prev = jnp.where(i == 0, -jnp.inf, m_ref[0, 0])
    cur = jnp.maximum(prev, blockmax)
    m_ref[0, 0] = cur

    @pl.when(i == nsteps - 1)
    def _():
        fp = jnp.sqrt(jnp.sum(pmap_ref[...] * pmap_ref[...], axis=2))
        pmax = jnp.max(fp, axis=1, keepdims=True)
        pm = fp / pmax
        pm_ref[...] = pm
        q_ref[...] = (1.0 - pm) * cur + EPS
        nest_ref[...] = jnp.sum(pm, axis=1)[None, :]


def _prep(prob_map, dis_matrix):
    blk = 512
    grid = NV // blk
    return pl.pallas_call(
        _prep_body,
        grid=(grid,),
        compiler_params=pltpu.CompilerParams(
            vmem_limit_bytes=100 * 1024 * 1024),
        in_specs=[
            pl.BlockSpec((B, NV, 4), lambda i: (0, 0, 0)),
            pl.BlockSpec((blk, NV), lambda i: (i, 0)),
        ],
        out_specs=[
            pl.BlockSpec((B, NV), lambda i: (0, 0)),
            pl.BlockSpec((B, NV), lambda i: (0, 0)),
            pl.BlockSpec((1, B), lambda i: (0, 0)),
            pl.BlockSpec((1, 1), lambda i: (0, 0), memory_space=pltpu.SMEM),
        ],
        out_shape=[
            jax.ShapeDtypeStruct((B, NV), jnp.float32),
            jax.ShapeDtypeStruct((B, NV), jnp.float32),
            jax.ShapeDtypeStruct((1, B), jnp.float32),
            jax.ShapeDtypeStruct((1, 1), jnp.float32),
        ],
    )(prob_map, dis_matrix)


_MESH = plsc.VectorSubcoreMesh(core_axis_name="c", subcore_axis_name="s")


def _fast_recip(x):
    xi = plsc.bitcast(x, jnp.int32)
    r = plsc.bitcast(jnp.full((16,), 0x7EF311C3, jnp.int32) - xi, jnp.float32)
    r = r * (2.0 - x * r)
    r = r * (2.0 - x * r)
    return r


@functools.partial(
    pl.kernel,
    mesh=_MESH,
    compiler_params=pltpu.CompilerParams(needs_layout_passes=False),
    out_type=[
        jax.ShapeDtypeStruct((NW, B * NG), jnp.float32),
        jax.ShapeDtypeStruct((NW, B * 16), jnp.float32),
    ],
    scratch_types=[
        pltpu.VMEM((CH, NV), jnp.float32),
        pltpu.VMEM((B * NG,), jnp.int32),
        pltpu.VMEM((B * RPW,), jnp.float32),
        pltpu.VMEM((B * RPW,), jnp.float32),
        pltpu.VMEM((B * NG,), jnp.float32),
        pltpu.VMEM((B * 16,), jnp.float32),
    ],
)
def _scmain(dis_hbm, gt_hbm, pm_hbm, q_hbm, colsum_out, t1_out,
            rowbuf, idxbuf, pmbuf, qbuf, colsum, t1buf):
    c = lax.axis_index("c")
    s = lax.axis_index("s")
    wid = c * 16 + s
    r0 = wid * RPW

    pltpu.sync_copy(gt_hbm, idxbuf)
    for b in range(B):
        pltpu.sync_copy(pm_hbm.at[b, pl.ds(r0, RPW)],
                        pmbuf.at[pl.ds(b * RPW, RPW)])
        pltpu.sync_copy(q_hbm.at[b, pl.ds(r0, RPW)],
                        qbuf.at[pl.ds(b * RPW, RPW)])

    zero16 = jnp.zeros((16,), jnp.float32)
    for j in range(B * NG // 16):
        colsum[pl.ds(j * 16, 16)] = zero16

    def chunk_body(ci, t1c):
        pltpu.sync_copy(dis_hbm.at[pl.ds(r0 + ci * CH, CH)], rowbuf)
        t1l = list(t1c)
        for b in range(B):
            cvecs = [idxbuf[pl.ds(b * NG + k * 16, 16)] for k in range(NG // 16)]

            def r_body(r, acc, b=b, cvecs=cvecs, ci=ci):
                lrow = ci * CH + r
                rsplat = jnp.full((16,), r, dtype=jnp.int32)
                pmoff = jnp.full((16,), b * RPW + lrow, dtype=jnp.int32)
                pmv = plsc.load_gather(pmbuf, [pmoff])
                qv = plsc.load_gather(qbuf, [pmoff])
                minacc = jnp.full((16,), jnp.inf, dtype=jnp.float32)
                for k in range(NG // 16):
                    g = plsc.load_gather(rowbuf, [rsplat, cvecs[k]])
                    minacc = jnp.minimum(minacc, g)
                    rec = _fast_recip(g * pmv + qv)
                    plsc.addupdate(colsum.at[pl.ds(b * NG + k * 16, 16)], rec)
                rowmin = jnp.min(minacc)
                return acc + pmv * rowmin

            t1l[b] = lax.fori_loop(0, CH, r_body, t1l[b])
        return tuple(t1l)

    t1 = lax.fori_loop(0, NCHUNK, chunk_body,
                       tuple(jnp.zeros((16,), jnp.float32) for _ in range(B)))

    for b in range(B):
        t1buf[pl.ds(b * 16, 16)] = t1[b] * (1.0 / 16.0)

    pltpu.sync_copy(colsum, colsum_out.at[wid])
    pltpu.sync_copy(t1buf, t1_out.at[wid])


def _epi_body(colsum_ref, t1_ref, nest_ref, out_ref):
    S = jnp.sum(colsum_ref[...], axis=0, keepdims=True)
    term2_sum = jnp.sum(float(NV) / S) * (1.0 / NG)
    t1acc = 0.0
    for b in range(B):
        t1num_b = jnp.sum(t1_ref[:, b * 16:(b + 1) * 16])
        t1acc = t1acc + t1num_b / (nest_ref[0, b] + EPS)
    out_ref[0, 0] = (t1acc + term2_sum) * (1.0 / B)


def _epi(colsum_part, t1_part, nest):
    return pl.pallas_call(
        _epi_body,
        in_specs=[
            pl.BlockSpec(memory_space=pltpu.VMEM),
            pl.BlockSpec(memory_space=pltpu.VMEM),
            pl.BlockSpec(memory_space=pltpu.SMEM),
        ],
        out_specs=pl.BlockSpec(memory_space=pltpu.SMEM),
        out_shape=jax.ShapeDtypeStruct((1, 1), jnp.float32),
    )(colsum_part, t1_part, nest)


def kernel(prob_map, gt, dis_matrix):
    pm, q, nest, _m = _prep(prob_map, dis_matrix)
    gt_flat = gt.reshape(-1)
    colsum_part, t1_part = _scmain(dis_matrix, gt_flat, pm, q)
    res = _epi(colsum_part, t1_part, nest)
    return res[0, 0]


# speedup vs baseline: 1.9776x; 1.9142x over previous
import functools

import jax
import jax.numpy as jnp
from jax import lax
from jax.experimental import pallas as pl
from jax.experimental.pallas import tpu as pltpu
from jax.experimental.pallas import tpu_sc as plsc

B = 8
NV = 4096
NG = 256
NW = 32
RPW = NV // NW
CH = 8
NCHUNK = RPW // CH
EPS = 1e-6


def _prep_body(pmap_ref, dis_ref, pm_ref, q_ref, nest_ref, m_ref):
    i = pl.program_id(0)
    nsteps = pl.num_programs(0)
    blockmax = jnp.max(dis_ref[...])
    prev = jnp.where(i == 0, -jnp.inf, m_ref[0, 0])
    cur = jnp.maximum(prev, blockmax)
    m_ref[0, 0] = cur

    @pl.when(i == nsteps - 1)
    def _():
        fp = jnp.sqrt(jnp.sum(pmap_ref[...] * pmap_ref[...], axis=2))
        pmax = jnp.max(fp, axis=1, keepdims=True)
        pm = fp / pmax
        pm_ref[...] = pm
        q_ref[...] = (1.0 - pm) * cur + EPS
        nest_ref[...] = jnp.sum(pm, axis=1)[None, :]


def _prep(prob_map, dis_matrix):
    blk = 512
    grid = NV // blk
    return pl.pallas_call(
        _prep_body,
        grid=(grid,),
        compiler_params=pltpu.CompilerParams(
            vmem_limit_bytes=100 * 1024 * 1024),
        in_specs=[
            pl.BlockSpec((B, NV, 4), lambda i: (0, 0, 0)),
            pl.BlockSpec((blk, NV), lambda i: (i, 0)),
        ],
        out_specs=[
            pl.BlockSpec((B, NV), lambda i: (0, 0)),
            pl.BlockSpec((B, NV), lambda i: (0, 0)),
            pl.BlockSpec((1, B), lambda i: (0, 0)),
            pl.BlockSpec((1, 1), lambda i: (0, 0), memory_space=pltpu.SMEM),
        ],
        out_shape=[
            jax.ShapeDtypeStruct((B, NV), jnp.float32),
            jax.ShapeDtypeStruct((B, NV), jnp.float32),
            jax.ShapeDtypeStruct((1, B), jnp.float32),
            jax.ShapeDtypeStruct((1, 1), jnp.float32),
        ],
    )(prob_map, dis_matrix)


_MESH = plsc.VectorSubcoreMesh(core_axis_name="c", subcore_axis_name="s")


def _fast_recip(x):
    xi = plsc.bitcast(x, jnp.int32)
    r = plsc.bitcast(jnp.full((16,), 0x7EF311C3, jnp.int32) - xi, jnp.float32)
    r = r * (2.0 - x * r)
    r = r * (2.0 - x * r)
    return r


@functools.partial(
    pl.kernel,
    mesh=_MESH,
    compiler_params=pltpu.CompilerParams(needs_layout_passes=False),
    out_type=[
        jax.ShapeDtypeStruct((NW, B * NG), jnp.float32),
        jax.ShapeDtypeStruct((NW, B * 16), jnp.float32),
    ],
    scratch_types=[
        pltpu.VMEM((CH, NV), jnp.float32),
        pltpu.VMEM((B * NG,), jnp.int32),
        pltpu.VMEM((B * RPW,), jnp.float32),
        pltpu.VMEM((B * RPW,), jnp.float32),
        pltpu.VMEM((B * NG,), jnp.float32),
        pltpu.VMEM((B * 16,), jnp.float32),
    ],
)
def _scmain(dis_hbm, gt_hbm, pm_hbm, q_hbm, colsum_out, t1_out,
            rowbuf, idxbuf, pmbuf, qbuf, colsum, t1buf):
    c = lax.axis_index("c")
    s = lax.axis_index("s")
    wid = c * 16 + s
    r0 = wid * RPW

    pltpu.sync_copy(gt_hbm, idxbuf)
    for b in range(B):
        pltpu.sync_copy(pm_hbm.at[b, pl.ds(r0, RPW)],
                        pmbuf.at[pl.ds(b * RPW, RPW)])
        pltpu.sync_copy(q_hbm.at[b, pl.ds(r0, RPW)],
                        qbuf.at[pl.ds(b * RPW, RPW)])

    zero16 = jnp.zeros((16,), jnp.float32)
    for j in range(B * NG // 16):
        colsum[pl.ds(j * 16, 16)] = zero16
    for b in range(B):
        t1buf[pl.ds(b * 16, 16)] = zero16

    rsplats = [jnp.full((16,), r, dtype=jnp.int32) for r in range(CH)]

    def chunk_body(ci, _):
        pltpu.sync_copy(dis_hbm.at[pl.ds(r0 + ci * CH, CH)], rowbuf)

        def b_body(b, __, ci=ci):
            pmoff0 = b * RPW + ci * CH
            pmvs = [plsc.load_gather(
                pmbuf, [jnp.full((16,), pmoff0 + r, jnp.int32)])
                for r in range(CH)]
            qvs = [plsc.load_gather(
                qbuf, [jnp.full((16,), pmoff0 + r, jnp.int32)])
                for r in range(CH)]
            minaccs = [jnp.full((16,), jnp.inf, jnp.float32) for _ in range(CH)]
            for k in range(NG // 16):
                cvec = idxbuf[pl.ds(b * NG + k * 16, 16)]
                colacc = zero16
                for r in range(CH):
                    g = plsc.load_gather(rowbuf, [rsplats[r], cvec])
                    minaccs[r] = jnp.minimum(minaccs[r], g)
                    colacc = colacc + _fast_recip(g * pmvs[r] + qvs[r])
                plsc.addupdate(colsum.at[pl.ds(b * NG + k * 16, 16)], colacc)
            t1c = zero16
            for r in range(CH):
                t1c = t1c + pmvs[r] * jnp.min(minaccs[r])
            plsc.addupdate(t1buf.at[pl.ds(b * 16, 16)], t1c * (1.0 / 16.0))
            return __

        return lax.fori_loop(0, B, b_body, _)

    lax.fori_loop(0, NCHUNK, chunk_body, jnp.int32(0))

    pltpu.sync_copy(colsum, colsum_out.at[wid])
    pltpu.sync_copy(t1buf, t1_out.at[wid])


def _epi_body(colsum_ref, t1_ref, nest_ref, out_ref):
    S = jnp.sum(colsum_ref[...], axis=0, keepdims=True)
    term2_sum = jnp.sum(float(NV) / S) * (1.0 / NG)
    t1acc = 0.0
    for b in range(B):
        t1num_b = jnp.sum(t1_ref[:, b * 16:(b + 1) * 16])
        t1acc = t1acc + t1num_b / (nest_ref[0, b] + EPS)
    out_ref[0, 0] = (t1acc + term2_sum) * (1.0 / B)


def _epi(colsum_part, t1_part, nest):
    return pl.pallas_call(
        _epi_body,
        in_specs=[
            pl.BlockSpec(memory_space=pltpu.VMEM),
            pl.BlockSpec(memory_space=pltpu.VMEM),
            pl.BlockSpec(memory_space=pltpu.SMEM),
        ],
        out_specs=pl.BlockSpec(memory_space=pltpu.SMEM),
        out_shape=jax.ShapeDtypeStruct((1, 1), jnp.float32),
    )(colsum_part, t1_part, nest)


def kernel(prob_map, gt, dis_matrix):
    pm, q, nest, _m = _prep(prob_map, dis_matrix)
    gt_flat = gt.reshape(-1)
    colsum_part, t1_part = _scmain(dis_matrix, gt_flat, pm, q)
    res = _epi(colsum_part, t1_part, nest)
    return res[0, 0]


# speedup vs baseline: 2.6441x; 1.3370x over previous
import functools

import jax
import jax.numpy as jnp
from jax import lax
from jax.experimental import pallas as pl
from jax.experimental.pallas import tpu as pltpu
from jax.experimental.pallas import tpu_sc as plsc

B = 8
NV = 4096
NG = 256
NC = B * NG
NW = 32
RPW = NV // NW
CH = 8
NCHUNK = RPW // CH
EPS = 1e-6


_MESH = plsc.VectorSubcoreMesh(core_axis_name="c", subcore_axis_name="s")


@functools.partial(
    pl.kernel,
    mesh=_MESH,
    compiler_params=pltpu.CompilerParams(needs_layout_passes=False),
    out_type=jax.ShapeDtypeStruct((NV, NC), jnp.float32),
    scratch_types=[
        pltpu.VMEM((CH, NV), jnp.float32),
        pltpu.VMEM((CH, NV), jnp.float32),
        pltpu.VMEM((CH, NC), jnp.float32),
        pltpu.VMEM((CH, NC), jnp.float32),
        pltpu.VMEM((NC,), jnp.int32),
        pltpu.SemaphoreType.DMA,
        pltpu.SemaphoreType.DMA,
        pltpu.SemaphoreType.DMA,
        pltpu.SemaphoreType.DMA,
    ],
)
def _scgather(dis_hbm, gt_hbm, g_out,
              rowbuf0, rowbuf1, stage0, stage1, idxbuf,
              insem0, insem1, outsem0, outsem1):
    c = lax.axis_index("c")
    s = lax.axis_index("s")
    wid = c * 16 + s
    r0 = wid * RPW

    pltpu.sync_copy(gt_hbm, idxbuf)

    rowbufs = (rowbuf0, rowbuf1)
    stages = (stage0, stage1)
    insems = (insem0, insem1)
    outsems = (outsem0, outsem1)
    rsplats = [jnp.full((16,), r, dtype=jnp.int32) for r in range(CH)]

    def start_in(ci, buf, sem):
        pltpu.async_copy(dis_hbm.at[pl.ds(r0 + ci * CH, CH)], buf, sem)

    start_in(0, rowbufs[0], insems[0])

    def chunk_pair(half, _):
        for p in range(2):
            ci = half * 2 + p
            nxt = (p + 1) % 2

            @pl.when(ci + 1 < NCHUNK)
            def _(ci=ci, nxt=nxt):
                start_in(ci + 1, rowbufs[nxt], insems[nxt])

            pltpu.make_async_copy(
                dis_hbm.at[pl.ds(r0 + ci * CH, CH)], rowbufs[p], insems[p]
            ).wait()
            @pl.when(ci >= 2)
            def _(ci=ci, p=p):
                pltpu.make_async_copy(
                    stages[p], g_out.at[pl.ds(r0 + (ci - 2) * CH, CH)],
                    outsems[p],
                ).wait()

            def k_body(k, __, p=p):
                cvec = idxbuf[pl.ds(k * 16, 16)]
                for r in range(CH):
                    g = plsc.load_gather(rowbufs[p], [rsplats[r], cvec])
                    stages[p][r, pl.ds(k * 16, 16)] = g
                return __

            lax.fori_loop(0, NC // 16, k_body, jnp.int32(0))
            pltpu.async_copy(
                stages[p], g_out.at[pl.ds(r0 + ci * CH, CH)], outsems[p])
        return jnp.int32(0)

    lax.fori_loop(0, NCHUNK // 2, chunk_pair, jnp.int32(0))

    for p in range(2):
        ci = NCHUNK - 2 + p
        pltpu.make_async_copy(
            stages[p], g_out.at[pl.ds(r0 + ci * CH, CH)], outsems[p]
        ).wait()


def _prep_body(pmap_ref, dis_ref, pmt_ref, qt_ref, nest_ref, m_ref):
    i = pl.program_id(0)
    nsteps = pl.num_programs(0)
    blockmax = jnp.max(dis_ref[...])
    prev = jnp.where(i == 0, -jnp.inf, m_ref[0, 0])
    cur = jnp.maximum(prev, blockmax)
    m_ref[0, 0] = cur

    @pl.when(i == nsteps - 1)
    def _():
        fp = jnp.sqrt(jnp.sum(pmap_ref[...] * pmap_ref[...], axis=2))
        pmax = jnp.max(fp, axis=1, keepdims=True)
        pm = fp / pmax
        pmt = pm.T
        pmt_ref[...] = pmt
        qt_ref[...] = (1.0 - pmt) * cur + EPS
        nest_ref[...] = jnp.sum(pm, axis=1)[None, :]


def _prep(prob_map, dis_matrix):
    blk = 512
    grid = NV // blk
    return pl.pallas_call(
        _prep_body,
        grid=(grid,),
        compiler_params=pltpu.CompilerParams(
            vmem_limit_bytes=100 * 1024 * 1024),
        in_specs=[
            pl.BlockSpec((B, NV, 4), lambda i: (0, 0, 0)),
            pl.BlockSpec((blk, NV), lambda i: (i, 0)),
        ],
        out_specs=[
            pl.BlockSpec((NV, B), lambda i: (0, 0)),
            pl.BlockSpec((NV, B), lambda i: (0, 0)),
            pl.BlockSpec((1, B), lambda i: (0, 0)),
            pl.BlockSpec((1, 1), lambda i: (0, 0), memory_space=pltpu.SMEM),
        ],
        out_shape=[
            jax.ShapeDtypeStruct((NV, B), jnp.float32),
            jax.ShapeDtypeStruct((NV, B), jnp.float32),
            jax.ShapeDtypeStruct((1, B), jnp.float32),
            jax.ShapeDtypeStruct((1, 1), jnp.float32),
        ],
    )(prob_map, dis_matrix)


_FBLK = 512
_FSTEPS = NV // _FBLK


def _final_body(g_ref, pmt_ref, qt_ref, nest_ref, out_ref, cs_ref, t1_ref):
    i = pl.program_id(0)

    pmt = pmt_ref[...]
    qt = qt_ref[...]
    pme = jnp.concatenate(
        [jnp.broadcast_to(pmt[:, b:b + 1], (_FBLK, NG)) for b in range(B)],
        axis=1)
    qe = jnp.concatenate(
        [jnp.broadcast_to(qt[:, b:b + 1], (_FBLK, NG)) for b in range(B)],
        axis=1)
    g = g_ref[...]
    rec = 1.0 / (g * pme + qe)
    cs_prev = jnp.where(i == 0, 0.0, cs_ref[...])
    cs_ref[...] = cs_prev + jnp.sum(rec, axis=0, keepdims=True)

    gmin = jnp.min(g.reshape(_FBLK, B, NG), axis=2)
    t1_prev = jnp.where(i == 0, 0.0, t1_ref[...])
    t1_ref[...] = t1_prev + jnp.sum(pmt * gmin, axis=0, keepdims=True)

    @pl.when(i == _FSTEPS - 1)
    def _():
        term2 = jnp.sum(float(NV) / cs_ref[...]) * (1.0 / (NG * B))
        term1 = jnp.sum(t1_ref[...] / (nest_ref[...] + EPS)) * (1.0 / B)
        out_ref[0, 0] = term1 + term2


def _final(g, pmt, qt, nest):
    return pl.pallas_call(
        _final_body,
        grid=(_FSTEPS,),
        in_specs=[
            pl.BlockSpec((_FBLK, NC), lambda i: (i, 0)),
            pl.BlockSpec((_FBLK, B), lambda i: (i, 0)),
            pl.BlockSpec((_FBLK, B), lambda i: (i, 0)),
            pl.BlockSpec((1, B), lambda i: (0, 0)),
        ],
        out_specs=pl.BlockSpec((1, 1), lambda i: (0, 0),
                               memory_space=pltpu.SMEM),
        out_shape=jax.ShapeDtypeStruct((1, 1), jnp.float32),
        scratch_shapes=[
            pltpu.VMEM((1, NC), jnp.float32),
            pltpu.VMEM((1, B), jnp.float32),
        ],
    )(g, pmt, qt, nest)


def kernel(prob_map, gt, dis_matrix):
    gt_flat = gt.reshape(-1)
    g = _scgather(dis_matrix, gt_flat)
    pmt, qt, nest, _m = _prep(prob_map, dis_matrix)
    res = _final(g, pmt, qt, nest)
    return res[0, 0]
